# Initial kernel scaffold; baseline (speedup 1.0000x reference)
import jax
import jax.numpy as jnp
from jax.experimental import pallas as pl


def kernel(coo, rel_pos, consistancy, edges, graph_id, soe, virt_dirs, W_node, b_node, relpos_table, W_edge, b_edge, W_att, W_v, W_o, W_m1, W_m2, W_g, W_e1, W_e2, W_soe, W_out, b_out):
    raise NotImplementedError("write your pallas kernel here")


# speedup vs baseline: 1.9788x; 1.9788x over previous
import jax
import jax.numpy as jnp
from jax import lax
from jax.experimental import pallas as pl
from jax.experimental.pallas import tpu as pltpu
from jax.experimental.pallas import tpu_sc as plsc

N = 10000
E = 160000
D = 128
H = 8
G = 64

NC = 2
NS = 16
LANE = 16

EPT_BIG = 5120

SOE_C = 10000
SOE_P = E // SOE_C
SOE_EPS = E // NS
SOE_SCHUNK = 2000
CROWS = (SOE_C + LANE + 127) // 128

_mesh = plsc.VectorSubcoreMesh(core_axis_name="c", subcore_axis_name="s",
                               num_cores=NC, num_subcores=NS)
_sc_params = pltpu.CompilerParams(needs_layout_passes=False)

_ZSZ = N // NS // 8 * 8
_ZLAST = N - (NS - 1) * _ZSZ


def _bcast(v, h):
  dn = lax.GatherDimensionNumbers(offset_dims=(), collapsed_slice_dims=(0,),
                                  start_index_map=(0,))
  return lax.gather(v, jnp.full((LANE, 1), h, jnp.int32), dn, (1,),
                    mode=lax.GatherScatterMode.PROMISE_IN_BOUNDS)


def _edge_partition():
  c = lax.axis_index("c")
  s = lax.axis_index("s")
  wid = c * NS + s
  return c, s, wid * EPT_BIG, jnp.where(wid < NC * NS - 1, EPT_BIG,
                                        E - (NC * NS - 1) * EPT_BIG)


def _spread(ref_src, ref_dst, s):
  @pl.when(s < NS - 1)
  def _():
    pltpu.sync_copy(ref_src.at[pl.ds(s * _ZSZ, _ZSZ)],
                    ref_dst.at[pl.ds(s * _ZSZ, _ZSZ)])

  @pl.when(s == NS - 1)
  def _():
    pltpu.sync_copy(ref_src.at[pl.ds((NS - 1) * _ZSZ, _ZLAST)],
                    ref_dst.at[pl.ds((NS - 1) * _ZSZ, _ZLAST)])


def _sc_coo_gather(src, dst, coo128):
  CH = 128

  def body(src_h, dst_h, tab_h, cs_o, cd_o, sidx, didx, rows_s, rows_d,
           out_s, out_d, sem):
    _, _, base0, ept = _edge_partition()

    def step(k, _):
      base = base0 + k * CH
      pltpu.sync_copy(src_h.at[pl.ds(base, CH)], sidx)
      pltpu.sync_copy(dst_h.at[pl.ds(base, CH)], didx)
      pltpu.async_copy(tab_h.at[sidx], rows_s, sem).wait()
      pltpu.async_copy(tab_h.at[didx], rows_d, sem).wait()

      def rep(i, _):
        out_s[i] = rows_s[i, pl.ds(0, LANE)]
        out_d[i] = rows_d[i, pl.ds(0, LANE)]
        return 0

      lax.fori_loop(0, CH, rep, 0)
      pltpu.sync_copy(out_s, cs_o.at[pl.ds(base, CH)])
      pltpu.sync_copy(out_d, cd_o.at[pl.ds(base, CH)])
      return 0

    lax.fori_loop(0, ept // CH, step, 0)

  k = pl.kernel(
      body,
      out_type=(jax.ShapeDtypeStruct((E, LANE), jnp.float32),
                jax.ShapeDtypeStruct((E, LANE), jnp.float32)),
      mesh=_mesh,
      compiler_params=_sc_params,
      scratch_types=[
          pltpu.VMEM((CH,), jnp.int32),
          pltpu.VMEM((CH,), jnp.int32),
          pltpu.VMEM((CH, 128), jnp.float32),
          pltpu.VMEM((CH, 128), jnp.float32),
          pltpu.VMEM((CH, LANE), jnp.float32),
          pltpu.VMEM((CH, LANE), jnp.float32),
          pltpu.SemaphoreType.DMA,
      ])
  return k(src, dst, coo128)


def _sc_gather2(src, dst, tab_s, tab_d):
  CH = 128

  def body(src_h, dst_h, ts_h, td_h, os_h, od_h, sidx, didx, rows_s, rows_d,
           sem):
    _, _, base0, ept = _edge_partition()

    def step(k, _):
      base = base0 + k * CH
      pltpu.sync_copy(src_h.at[pl.ds(base, CH)], sidx)
      pltpu.sync_copy(dst_h.at[pl.ds(base, CH)], didx)
      pltpu.async_copy(ts_h.at[sidx], rows_s, sem).wait()
      pltpu.sync_copy(rows_s, os_h.at[pl.ds(base, CH)])
      pltpu.async_copy(td_h.at[didx], rows_d, sem).wait()
      pltpu.sync_copy(rows_d, od_h.at[pl.ds(base, CH)])
      return 0

    lax.fori_loop(0, ept // CH, step, 0)

  k = pl.kernel(
      body,
      out_type=(jax.ShapeDtypeStruct((E, D), jnp.float32),
                jax.ShapeDtypeStruct((E, D), jnp.float32)),
      mesh=_mesh,
      compiler_params=_sc_params,
      scratch_types=[
          pltpu.VMEM((CH,), jnp.int32),
          pltpu.VMEM((CH,), jnp.int32),
          pltpu.VMEM((CH, D), jnp.float32),
          pltpu.VMEM((CH, D), jnp.float32),
          pltpu.SemaphoreType.DMA,
      ])
  return k(src, dst, tab_s, tab_d)


def _sc_attn(src, dst, ae, tsrc, tdst, zeros_h_in):
  CH = 64

  def body(src_h, dst_h, ae_h, tsrc_h, tdst_h, zeros_h, num_h, den_h, exm_h,
           acc, sidx, didx, rows, adrow, aebuf, contrib, exmbuf, sem):
    c, s, base0, ept = _edge_partition()
    lane = lax.broadcasted_iota(jnp.int32, (LANE,), 0)

    _spread(zeros_h, acc, s)
    plsc.subcore_barrier()

    def step_a(k, _):
      base = base0 + k * CH
      pltpu.sync_copy(src_h.at[pl.ds(base, CH)], sidx)
      pltpu.sync_copy(dst_h.at[pl.ds(base, CH)], didx)
      pltpu.async_copy(tsrc_h.at[sidx], rows, sem).wait()
      pltpu.async_copy(tdst_h.at[didx], adrow, sem).wait()
      pltpu.sync_copy(ae_h.at[pl.ds(base, CH)], aebuf)

      def edge(i, _):
        asv = rows[i, pl.ds(D, LANE)]
        adv = adrow[i, pl.ds(0, LANE)]
        aev = aebuf[i]
        ex = jnp.exp(asv + adv + aev)
        exmbuf[i] = jnp.where(lane < H, ex, 0.0)
        for hh in range(H):
          contrib[i, pl.ds(hh * LANE, LANE)] = (
              rows[i, pl.ds(hh * LANE, LANE)] * _bcast(ex, hh))
        return 0

      lax.fori_loop(0, CH, edge, 0)
      pltpu.sync_copy(contrib, acc.at[didx], add=True)
      pltpu.sync_copy(exmbuf, exm_h.at[pl.ds(base, CH)])
      return 0

    lax.fori_loop(0, ept // CH, step_a, 0)
    plsc.subcore_barrier()
    _spread(acc, num_h.at[c], s)
    plsc.subcore_barrier()
    _spread(zeros_h, acc, s)
    plsc.subcore_barrier()

    def step_b(k, _):
      base = base0 + k * CH
      pltpu.sync_copy(dst_h.at[pl.ds(base, CH)], didx)
      pltpu.sync_copy(exm_h.at[pl.ds(base, CH)], aebuf)

      def edge(i, _):
        ex = aebuf[i]
        for hh in range(H):
          contrib[i, pl.ds(hh * LANE, LANE)] = _bcast(ex, hh)
        return 0

      lax.fori_loop(0, CH, edge, 0)
      pltpu.sync_copy(contrib, acc.at[didx], add=True)
      return 0

    lax.fori_loop(0, ept // CH, step_b, 0)
    plsc.subcore_barrier()
    _spread(acc, den_h.at[c], s)

  k = pl.kernel(
      body,
      out_type=(jax.ShapeDtypeStruct((NC, N, D), jnp.float32),
                jax.ShapeDtypeStruct((NC, N, D), jnp.float32),
                jax.ShapeDtypeStruct((E, LANE), jnp.float32)),
      mesh=_mesh,
      compiler_params=_sc_params,
      scratch_types=[
          pltpu.VMEM_SHARED((N, D), jnp.float32),
          pltpu.VMEM((CH,), jnp.int32),
          pltpu.VMEM((CH,), jnp.int32),
          pltpu.VMEM((CH, 2 * D), jnp.float32),
          pltpu.VMEM((CH, D), jnp.float32),
          pltpu.VMEM((CH, LANE), jnp.float32),
          pltpu.VMEM((CH, D), jnp.float32),
          pltpu.VMEM((CH, LANE), jnp.float32),
          pltpu.SemaphoreType.DMA,
      ])
  return k(src, dst, ae, tsrc, tdst, zeros_h_in)


def _sc_soe(soe0, soe1, e2, zeros_h_in):
  NSC = SOE_EPS // SOE_SCHUNK
  NV = SOE_SCHUNK // LANE

  def body(soe0_h, soe1_h, e2_h, zeros_h, sagg_h, cnt_h,
           acc, s0c, s1c, cntl, gbuf, lbuf, gfire, lfire, rows, sem):
    c = lax.axis_index("c")
    s = lax.axis_index("s")
    lane = lax.broadcasted_iota(jnp.int32, (LANE,), 0)
    ones = jnp.full((LANE,), 1.0, jnp.float32)
    zv16 = jnp.zeros((LANE,), jnp.float32)

    def count_fire(nvalid):
      def cedge(j, _):
        lv = lfire[pl.ds((j // LANE) * LANE, LANE)]
        db = _bcast(lv, j % LANE)
        plsc.addupdate_scatter(cntl, [db // 128, db % 128], ones,
                               mask=(lane == 0) & (j < nvalid))
        return 0

      lax.fori_loop(0, 128, cedge, 0)

    def fire_full(cn):
      for j in range(128 // LANE):
        gfire[pl.ds(j * LANE, LANE)] = gbuf[pl.ds(j * LANE, LANE)]
        lfire[pl.ds(j * LANE, LANE)] = lbuf[pl.ds(j * LANE, LANE)]
      pltpu.async_copy(e2_h.at[gfire], rows, sem).wait()
      pltpu.sync_copy(rows, acc.at[lfire], add=True)
      count_fire(jnp.int32(128))
      gt = gbuf[pl.ds(128, LANE)]
      lt = lbuf[pl.ds(128, LANE)]
      gbuf[pl.ds(0, LANE)] = gt
      lbuf[pl.ds(0, LANE)] = lt
      return cn - 128

    def do_pass(p, _):
      @pl.when(p % NC == c)
      def _():
        lo = p * SOE_C
        _spread(zeros_h, acc, s)

        def zc(v, _):
          cntl[v // 8, pl.ds((v % 8) * LANE, LANE)] = zv16
          return 0

        lax.fori_loop(0, CROWS * 8, zc, 0)
        plsc.subcore_barrier()

        def scan_chunk(q, cnt):
          sb = s * SOE_EPS + q * SOE_SCHUNK
          pltpu.sync_copy(soe0_h.at[pl.ds(sb, SOE_SCHUNK)], s0c)
          pltpu.sync_copy(soe1_h.at[pl.ds(sb, SOE_SCHUNK)], s1c)

          def scan(v, cnt):
            off = v * LANE
            dv = s1c[pl.ds(off, LANE)]
            gv = s0c[pl.ds(off, LANE)]
            m = (dv >= lo) & (dv < lo + SOE_C)
            plsc.store_compressed(gbuf.at[pl.ds(cnt, LANE)], gv, mask=m)
            plsc.store_compressed(lbuf.at[pl.ds(cnt, LANE)], dv - lo, mask=m)
            cnt = cnt + jnp.sum(m.astype(jnp.int32))
            return lax.cond(cnt >= 128, fire_full, lambda x: x, cnt)

          return lax.fori_loop(0, NV, scan, cnt)

        cnt = lax.fori_loop(0, NSC, scan_chunk, jnp.int32(0))
        for j in range(128 // LANE):
          keep = (lane + j * LANE) < cnt
          gfire[pl.ds(j * LANE, LANE)] = jnp.where(
              keep, gbuf[pl.ds(j * LANE, LANE)], 0)
          lfire[pl.ds(j * LANE, LANE)] = jnp.where(
              keep, lbuf[pl.ds(j * LANE, LANE)], SOE_C)
        pltpu.async_copy(e2_h.at[gfire], rows, sem).wait()
        pltpu.sync_copy(rows, acc.at[lfire], add=True)
        count_fire(cnt)
        plsc.subcore_barrier()
        _spread(acc, sagg_h.at[pl.ds(lo, SOE_C)], s)
        pltpu.sync_copy(cntl, cnt_h.at[s, p])
        plsc.subcore_barrier()
      return 0

    lax.fori_loop(0, SOE_P, do_pass, 0)

  k = pl.kernel(
      body,
      out_type=(jax.ShapeDtypeStruct((E, D), jnp.float32),
                jax.ShapeDtypeStruct((NS, SOE_P, CROWS, 128), jnp.float32)),
      mesh=_mesh,
      compiler_params=_sc_params,
      scratch_types=[
          pltpu.VMEM_SHARED((SOE_C + LANE, D), jnp.float32),
          pltpu.VMEM((SOE_SCHUNK,), jnp.int32),
          pltpu.VMEM((SOE_SCHUNK,), jnp.int32),
          pltpu.VMEM((CROWS, 128), jnp.float32),
          pltpu.VMEM((128 + 2 * LANE,), jnp.int32),
          pltpu.VMEM((128 + 2 * LANE,), jnp.int32),
          pltpu.VMEM((128,), jnp.int32),
          pltpu.VMEM((128,), jnp.int32),
          pltpu.VMEM((128, D), jnp.float32),
          pltpu.SemaphoreType.DMA,
      ])
  return k(soe0, soe1, e2, zeros_h_in)


def _mm(a, b):
  return lax.dot_general(a, b, (((a.ndim - 1,), (0,)), ((), ())),
                         precision=lax.Precision.HIGHEST)


def _ln(x):
  m = jnp.mean(x, -1, keepdims=True)
  v = jnp.var(x, -1, keepdims=True)
  return (x - m) * lax.rsqrt(v + 1e-5)


def _tc_pre(coo128, a128, r1, wv, was, wad):
  def body(coo_r, a_r, r_r, wv_r, was_r, wad_r, h0_o, tsrc_o, tdst_o):
    h0 = _mm(coo_r[...], a_r[...]) + r_r[...]
    h0_o[...] = h0
    tsrc_o[...] = jnp.concatenate(
        [_mm(h0, wv_r[...]), _mm(h0, was_r[...]), jnp.zeros((N, 120), jnp.float32)], 1)
    tdst_o[...] = jnp.concatenate(
        [_mm(h0, wad_r[...]), jnp.zeros((N, 120), jnp.float32)], 1)

  return pl.pallas_call(
      body,
      out_shape=(jax.ShapeDtypeStruct((N, D), jnp.float32),
                 jax.ShapeDtypeStruct((N, 2 * D), jnp.float32),
                 jax.ShapeDtypeStruct((N, D), jnp.float32)),
  )(coo128, a128, r1, wv, was, wad)


def _tc_edgefeat(cs, cd, rp2, cons, rpt, we_rbf, we_rp, we_c, be1, wae):
  BE = 2000

  def body(cs_r, cd_r, rp_r, cons_r, rpt_r, wrbf_r, wrp_r, wc_r, be_r, wae_r,
           e0_o, ae_o):
    diff = cd_r[:, 0:3] - cs_r[:, 0:3] + 1e-8
    dist = jnp.sqrt(jnp.sum(diff * diff, axis=1, keepdims=True))
    centers = lax.broadcasted_iota(jnp.int32, (1, 16), 1).astype(
        jnp.float32) * (20.0 / 15.0)
    rbf = jnp.exp(-0.5 * (dist - centers) ** 2)
    oh = (rp_r[...] == lax.broadcasted_iota(jnp.int32, (BE, 66), 1)
          ).astype(jnp.float32)
    rp16 = _mm(oh, rpt_r[...])
    e0 = (_mm(rbf, wrbf_r[...]) + _mm(rp16, wrp_r[...])
          + cons_r[...] * wc_r[...] + be_r[...])
    e0_o[...] = e0
    ae_o[...] = jnp.concatenate(
        [_mm(e0, wae_r[...]), jnp.zeros((BE, 8), jnp.float32)], 1)

  grid = (E // BE,)
  eb = lambda w: pl.BlockSpec((BE, w), lambda i: (i, 0))
  wb = lambda sh: pl.BlockSpec(sh, lambda i: (0, 0))
  return pl.pallas_call(
      body, grid=grid,
      in_specs=[eb(LANE), eb(LANE), eb(1), eb(1), wb((66, 16)),
                wb((16, D)), wb((16, D)), wb((1, D)), wb((1, D)), wb((D, H))],
      out_specs=(eb(D), eb(LANE)),
      out_shape=(jax.ShapeDtypeStruct((E, D), jnp.float32),
                 jax.ShapeDtypeStruct((E, LANE), jnp.float32)),
  )(cs, cd, rp2, cons, rpt, we_rbf, we_rp, we_c, be1, wae)


def _tc_node_mlp(h, pnum, pden, wo, wm1, wm2):
  BN = 2000

  def body(h_r, pn_r, pd_r, wo_r, wm1_r, wm2_r, h2_o):
    num = pn_r[0] + pn_r[1]
    den = pd_r[0] + pd_r[1]
    agg = num / (den + 1e-9)
    h1 = _ln(h_r[...] + _mm(agg, wo_r[...]))
    h2_o[...] = _ln(h1 + _mm(jax.nn.relu(_mm(h1, wm1_r[...])), wm2_r[...]))

  grid = (N // BN,)
  nb = pl.BlockSpec((BN, D), lambda i: (i, 0))
  pb = pl.BlockSpec((NC, BN, D), lambda i: (0, i, 0))
  wb = lambda sh: pl.BlockSpec(sh, lambda i: (0, 0))
  return pl.pallas_call(
      body, grid=grid,
      in_specs=[nb, pb, pb, wb((D, D)), wb((D, 2 * D)), wb((2 * D, D))],
      out_specs=nb,
      out_shape=jax.ShapeDtypeStruct((N, D), jnp.float32),
  )(h, pnum, pden, wo, wm1, wm2)


def _tc_graph_mean(h2, gid, wg):
  def body(h2_r, gid_r, wg_r, gp_o):
    oh = (gid_r[...] == lax.broadcasted_iota(jnp.int32, (N, G), 1)
          ).astype(jnp.float32)
    dn = (((0,), (0,)), ((), ()))
    cnt = lax.dot_general(oh, jnp.ones((N, 8), jnp.float32), dn,
                          precision=lax.Precision.HIGHEST)[:, 0:1]
    gm = lax.dot_general(oh, h2_r[...], dn, precision=lax.Precision.HIGHEST)
    gp_o[...] = _mm(gm / (cnt + 1e-6), wg_r[...])

  return pl.pallas_call(
      body, out_shape=jax.ShapeDtypeStruct((G, D), jnp.float32),
  )(h2, gid, wg)


def _tc_node_tables(h2, gid, gpro, extras, last):
  BN = 2000

  def body(*refs):
    if last:
      (h2_r, gid_r, gp_r, wout_r, bout_r, out_o) = refs
    else:
      (h2_r, gid_r, gp_r, we1s_r, we1d_r, wv_r, was_r, wad_r,
       h3_o, ts_o, td_o, tsrc_o, tdst_o) = refs
    oh = (gid_r[...] == lax.broadcasted_iota(jnp.int32, (BN, G), 1)
          ).astype(jnp.float32)
    h3 = h2_r[...] + _mm(oh, gp_r[...])
    if last:
      out_o[...] = _mm(h3, wout_r[...]) + bout_r[...]
    else:
      h3_o[...] = h3
      ts_o[...] = _mm(h3, we1s_r[...])
      td_o[...] = _mm(h3, we1d_r[...])
      z = jnp.zeros((BN, 120), jnp.float32)
      tsrc_o[...] = jnp.concatenate([_mm(h3, wv_r[...]), _mm(h3, was_r[...]),
                                     z], 1)
      tdst_o[...] = jnp.concatenate([_mm(h3, wad_r[...]), z], 1)

  grid = (N // BN,)
  nb = lambda w: pl.BlockSpec((BN, w), lambda i: (i, 0))
  wb = lambda sh: pl.BlockSpec(sh, lambda i: (0, 0))
  if last:
    out_shape = jax.ShapeDtypeStruct((N, 20), jnp.float32)
    out_specs = nb(20)
    in_specs = [nb(D), nb(1), wb((G, D)), wb((D, 20)), wb((1, 20))]
  else:
    out_shape = (jax.ShapeDtypeStruct((N, D), jnp.float32),
                 jax.ShapeDtypeStruct((N, D), jnp.float32),
                 jax.ShapeDtypeStruct((N, D), jnp.float32),
                 jax.ShapeDtypeStruct((N, 2 * D), jnp.float32),
                 jax.ShapeDtypeStruct((N, D), jnp.float32))
    out_specs = (nb(D), nb(D), nb(D), nb(2 * D), nb(D))
    in_specs = [nb(D), nb(1), wb((G, D)), wb((D, D)), wb((D, D)),
                wb((D, D)), wb((D, H)), wb((D, H))]
  return pl.pallas_call(body, grid=grid, in_specs=in_specs,
                        out_specs=out_specs, out_shape=out_shape)(
                            h2, gid, gpro, *extras)


def _tc_node_graph(h2, gid, wg, extras, last):
  gpro = _tc_graph_mean(h2, gid, wg)
  return _tc_node_tables(h2, gid, gpro, extras, last)


def _tc_epost(e0, tsg, tdg, we1e, we2):
  BE = 2000

  def body(e0_r, tsg_r, tdg_r, we1e_r, we2_r, e2_o):
    e0v = e0_r[...]
    u = jax.nn.relu(tsg_r[...] + tdg_r[...] + _mm(e0v, we1e_r[...]))
    e2_o[...] = _ln(e0v + _mm(u, we2_r[...]))

  grid = (E // BE,)
  eb = pl.BlockSpec((BE, D), lambda i: (i, 0))
  wb = pl.BlockSpec((D, D), lambda i: (0, 0))
  return pl.pallas_call(
      body, grid=grid,
      in_specs=[eb, eb, eb, wb, wb],
      out_specs=eb,
      out_shape=jax.ShapeDtypeStruct((E, D), jnp.float32),
  )(e0, tsg, tdg, we1e, we2)


def _tc_soe_fin(e2, sagg, cnts, wae1, wx):
  BE = 2000

  def body(e2_r, sag_r, cnt_r, wae_r, wx_r, ae_o):
    scnt = jnp.sum(cnt_r[...], axis=1, keepdims=True)
    ae1 = _mm(e2_r[...], wae_r[...]) + _mm(sag_r[...] / (scnt + 1e-6), wx_r[...])
    ae_o[...] = jnp.concatenate([ae1, jnp.zeros((BE, 8), jnp.float32)], 1)

  grid = (E // BE,)
  eb = lambda w: pl.BlockSpec((BE, w), lambda i: (i, 0))
  cb = pl.BlockSpec((BE, NS), lambda i: (i, 0))
  wb = pl.BlockSpec((D, H), lambda i: (0, 0))
  return pl.pallas_call(
      body, grid=grid,
      in_specs=[eb(D), eb(D), cb, wb, wb],
      out_specs=eb(LANE),
      out_shape=jax.ShapeDtypeStruct((E, LANE), jnp.float32),
  )(e2, sagg, cnts, wae1, wx)


def kernel(coo, rel_pos, consistancy, edges, graph_id, soe, virt_dirs,
           W_node, b_node, relpos_table, W_edge, b_edge, W_att, W_v, W_o,
           W_m1, W_m2, W_g, W_e1, W_e2, W_soe, W_out, b_out):
  f32 = jnp.float32
  src = edges[0].astype(jnp.int32)
  dst = edges[1].astype(jnp.int32)
  soe0 = soe[0].astype(jnp.int32)
  soe1 = soe[1].astype(jnp.int32)
  rp2 = rel_pos.astype(jnp.int32)[:, None]
  gid = graph_id.astype(jnp.int32)[:, None]
  cons = consistancy[:, None]

  coo128 = jnp.pad(coo.astype(f32), ((0, 0), (0, 125)))
  A = W_node[0:3] + W_node[3:6] + W_node[6:9] + W_node[9:12] + W_node[12:15]
  a128 = jnp.pad(A, ((0, 125), (0, 0)))
  r1 = (virt_dirs[0] @ W_node[3:6] + virt_dirs[1] @ W_node[6:9]
        + virt_dirs[2] @ W_node[9:12] + virt_dirs[3] @ W_node[12:15]
        + b_node)[None, :]
  was = [W_att[l][0:D] for l in range(2)]
  wad = [W_att[l][D:2 * D] for l in range(2)]
  wae = [W_att[l][2 * D:] for l in range(2)]
  we1s = [W_e1[l][0:D] for l in range(2)]
  we1d = [W_e1[l][D:2 * D] for l in range(2)]
  we1e = [W_e1[l][2 * D:] for l in range(2)]
  wx = W_soe[0] @ wae[1]
  zeros_n = jnp.zeros((N, D), f32)

  h0, tsrc0, tdst0 = _tc_pre(coo128, a128, r1, W_v[0], was[0], wad[0])
  cs, cd = _sc_coo_gather(src, dst, coo128)
  e0, ae0 = _tc_edgefeat(cs, cd, rp2, cons, relpos_table,
                         W_edge[0:16], W_edge[16:32], W_edge[32][None, :],
                         b_edge[None, :], wae[0])

  pn0, pd0, _ = _sc_attn(src, dst, ae0, tsrc0, tdst0, zeros_n)
  h2a = _tc_node_mlp(h0, pn0, pd0, W_o[0], W_m1[0], W_m2[0])
  h3, ts0, td0, tsrc1, tdst1 = _tc_node_graph(
      h2a, gid, W_g[0], (we1s[0], we1d[0], W_v[1], was[1], wad[1]),
      last=False)

  tsg, tdg = _sc_gather2(src, dst, ts0, td0)
  e2 = _tc_epost(e0, tsg, tdg, we1e[0], W_e2[0])
  sagg, cnts = _sc_soe(soe0, soe1, e2, zeros_n)
  cnts2 = (cnts.reshape(NS, SOE_P, CROWS * 128)[:, :, :SOE_C]
           .transpose(1, 2, 0).reshape(E, NS))
  ae1 = _tc_soe_fin(e2, sagg, cnts2, wae[1], wx)

  pn1, pd1, _ = _sc_attn(src, dst, ae1, tsrc1, tdst1, zeros_n)
  h2b = _tc_node_mlp(h3, pn1, pd1, W_o[1], W_m1[1], W_m2[1])
  out = _tc_node_graph(h2b, gid, W_g[1], (W_out, b_out[None, :]), last=True)
  return out
